# baseline (speedup 1.0000x reference)
import functools

import numpy as np

import jax
import jax.numpy as jnp
from jax import lax
from jax.experimental import pallas as pl
from jax.experimental.pallas import tpu as pltpu
from jax.experimental.pallas import tpu_sc as plsc

NEG_RATIO = 3
SIGN = np.uint32(0x80000000)
FMAX = np.float32(np.finfo(np.float32).max)

B, A, C = 128, 8732, 21
A_PAD = 8832
NW = 32
ROWS_PER_W = B // NW


def _keys_from_masked(masked):
    b = lax.bitcast_convert_type(masked, jnp.uint32)
    return jnp.where(b >= SIGN, ~b, b | SIGN)


def _vals_from_keys(u):
    b = jnp.where(u >= SIGN, u ^ SIGN, ~u)
    return lax.bitcast_convert_type(b, jnp.float32)


def _sc_gather_body(table_hbm, lt_hbm, out_hbm, lt_v, idx_v, val_v, sem):
    wid = lax.axis_index("s") * 2 + lax.axis_index("c")
    for i in range(ROWS_PER_W):
        b = wid * ROWS_PER_W + i
        pltpu.sync_copy(lt_hbm.at[b], lt_v)

        def chunk(j, _):
            lt16 = lt_v[pl.ds(j * 16, 16)]
            a = j * 16 + lax.iota(jnp.int32, 16)
            idx_v[pl.ds(j * 16, 16)] = b * (C * A) + lt16 * A + a
            return 0

        lax.fori_loop(0, A_PAD // 16, chunk, 0)
        pltpu.async_copy(table_hbm.at[idx_v], val_v, sem).wait()
        pltpu.sync_copy(val_v, out_hbm.at[b])


def _sc_gather(label_flat, lt_pad):
    k = pl.kernel(
        _sc_gather_body,
        out_type=jax.ShapeDtypeStruct((B, A_PAD), jnp.float32),
        scratch_types=[
            pltpu.VMEM((A_PAD,), jnp.int32),
            pltpu.VMEM((A_PAD,), jnp.int32),
            pltpu.VMEM((A_PAD,), jnp.float32),
            pltpu.SemaphoreType.DMA,
        ],
        mesh=plsc.VectorSubcoreMesh(core_axis_name="c", subcore_axis_name="s"),
    )
    return k(label_flat, lt_pad)


def _stage1_body(bb_in_ref, bb_tg_ref, g_ref, lt_ref, keys_ref, acc_ref):
    b = pl.program_id(0)

    lt = lt_ref[0]
    posf = (lt > 0).astype(jnp.float32)
    npos = jnp.sum(posf)

    d = bb_in_ref[0] - bb_tg_ref[0]
    ad = jnp.abs(d)
    sl1 = jnp.where(ad < 1.0, 0.5 * d * d, ad - 0.5)
    sl1_pos = jnp.sum(sl1 * posf[:, :A])

    label_loss = -g_ref[0]
    pos_ll = jnp.sum(label_loss * posf)

    masked = label_loss * (posf - 1.0)
    aidx = lax.broadcasted_iota(jnp.int32, (1, A_PAD), 1)
    masked = jnp.where(aidx < A, masked, FMAX)
    keys_ref[...] = _keys_from_masked(masked)[:, None, :]

    lane = lax.broadcasted_iota(jnp.int32, (1, 128), 1)
    contrib = jnp.where(lane == 0, sl1_pos,
                        jnp.where(lane == 1, pos_ll,
                                  jnp.where(lane == 2, npos, 0.0)))

    @pl.when(b == 0)
    def _():
        acc_ref[...] = jnp.zeros_like(acc_ref)

    acc_ref[...] += contrib


def _stage2_body(keys_ref, lt_ref, out_ref):
    u = keys_ref[:, 0, :]
    npos = jnp.sum((lt_ref[:, 0, :] > 0).astype(jnp.int32), axis=1,
                   keepdims=True)
    kv = jnp.minimum(NEG_RATIO * npos, A)

    def step(i, p):
        mid = p | (jnp.uint32(1) << (jnp.uint32(31) - i.astype(jnp.uint32)))
        cnt = jnp.sum((u < mid).astype(jnp.int32), axis=1, keepdims=True)
        return jnp.where(cnt >= kv, p, mid)

    p = lax.fori_loop(0, 32, step, jnp.zeros_like(kv, dtype=jnp.uint32))

    ltm = u < p
    c_lt = jnp.sum(ltm.astype(jnp.int32), axis=1, keepdims=True)
    masked = _vals_from_keys(u)
    sum_lt = jnp.sum(jnp.where(ltm, masked, 0.0), axis=1, keepdims=True)
    thr = _vals_from_keys(p)
    row_sel = sum_lt + (kv - c_lt).astype(jnp.float32) * thr
    row_sel = jnp.where(kv > 0, row_sel, 0.0)

    lane = lax.broadcasted_iota(jnp.int32, (1, 128), 1)
    out_ref[...] = jnp.where(lane == 0, jnp.sum(row_sel), 0.0)


@jax.jit
def kernel(bbox_input, label_input, bbox_target, label_target):
    lt = label_target.astype(jnp.int32)
    lt_pad = jnp.pad(lt, ((0, 0), (0, A_PAD - A)))
    bb_in = jnp.transpose(bbox_input, (0, 2, 1))
    bb_tg = jnp.transpose(bbox_target, (0, 2, 1))

    gathered = _sc_gather(label_input.reshape(-1), lt_pad)

    lt3 = lt_pad.reshape(B, 1, A_PAD)
    keys, acc = pl.pallas_call(
        _stage1_body,
        grid=(B,),
        in_specs=[
            pl.BlockSpec((1, 4, A), lambda b: (b, 0, 0)),
            pl.BlockSpec((1, 4, A), lambda b: (b, 0, 0)),
            pl.BlockSpec((1, 1, A_PAD), lambda b: (b, 0, 0)),
            pl.BlockSpec((1, 1, A_PAD), lambda b: (b, 0, 0)),
        ],
        out_specs=[
            pl.BlockSpec((1, 1, A_PAD), lambda b: (b, 0, 0)),
            pl.BlockSpec((1, 128), lambda b: (0, 0)),
        ],
        out_shape=[
            jax.ShapeDtypeStruct((B, 1, A_PAD), jnp.uint32),
            jax.ShapeDtypeStruct((1, 128), jnp.float32),
        ],
        compiler_params=pltpu.CompilerParams(
            dimension_semantics=("arbitrary",),
        ),
    )(bb_in, bb_tg, gathered.reshape(B, 1, A_PAD), lt3)

    sel = pl.pallas_call(
        _stage2_body,
        in_specs=[
            pl.BlockSpec((B, 1, A_PAD), lambda: (0, 0, 0)),
            pl.BlockSpec((B, 1, A_PAD), lambda: (0, 0, 0)),
        ],
        out_specs=pl.BlockSpec((1, 128), lambda: (0, 0)),
        out_shape=jax.ShapeDtypeStruct((1, 128), jnp.float32),
    )(keys, lt3)

    sl1_pos, pos_ll, npos = acc[0, 0], acc[0, 1], acc[0, 2]
    return (sl1_pos + pos_ll - sel[0, 0]) / npos


# speedup vs baseline: 5.7816x; 5.7816x over previous
import functools

import numpy as np

import jax
import jax.numpy as jnp
from jax import lax
from jax.experimental import pallas as pl
from jax.experimental.pallas import tpu as pltpu

NEG_RATIO = 3
SIGN = np.uint32(0x80000000)
RB = 8


def _keys_from_masked(masked):
    b = lax.bitcast_convert_type(masked, jnp.uint32)
    return jnp.where(b >= SIGN, ~b, b | SIGN)


def _vals_from_keys(u):
    b = jnp.where(u >= SIGN, u ^ SIGN, ~u)
    return lax.bitcast_convert_type(b, jnp.float32)


def _stage1_body(bb_in_ref, bb_tg_ref, li_ref, lt_ref, keys_ref, npr_ref,
                 acc_ref):
    step = pl.program_id(0)
    C, A = li_ref.shape[1], li_ref.shape[2]

    lt = lt_ref[...]
    posf = (lt > 0).astype(jnp.float32)
    npr = jnp.sum(posf, axis=1, keepdims=True)

    d = bb_in_ref[...] - bb_tg_ref[...]
    ad = jnp.abs(d)
    sl1 = jnp.where(ad < 1.0, 0.5 * d * d, ad - 0.5)
    sl1_pos = jnp.sum(sl1 * posf[:, None, :])

    li = li_ref[...]
    cid = lax.broadcasted_iota(jnp.int32, (RB, C, A), 1)
    onehot = (cid == lt[:, None, :]).astype(jnp.float32)
    label_loss = -jnp.sum(li * onehot, axis=1)
    pos_ll = jnp.sum(label_loss * posf)

    masked = label_loss * (posf - 1.0)
    keys_ref[...] = _keys_from_masked(masked)

    lane = lax.broadcasted_iota(jnp.int32, (RB, 128), 1)
    npr_ref[...] = jnp.where(lane == 0, npr, 0.0)

    lane1 = lax.broadcasted_iota(jnp.int32, (1, 128), 1)
    contrib = jnp.where(lane1 == 0, sl1_pos,
                        jnp.where(lane1 == 1, pos_ll,
                                  jnp.where(lane1 == 2, jnp.sum(npr), 0.0)))

    @pl.when(step == 0)
    def _():
        acc_ref[...] = jnp.zeros_like(acc_ref)

    acc_ref[...] += contrib


def _stage2_body(keys_ref, npr_ref, out_ref):
    A = keys_ref.shape[1]
    u = keys_ref[...]
    npos = npr_ref[:, 0:1].astype(jnp.int32)
    kv = jnp.minimum(NEG_RATIO * npos, A)

    def step(i, p):
        mid = p | (jnp.uint32(1) << (jnp.uint32(31) - i.astype(jnp.uint32)))
        cnt = jnp.sum((u < mid).astype(jnp.int32), axis=1, keepdims=True)
        return jnp.where(cnt >= kv, p, mid)

    p = lax.fori_loop(0, 32, step, jnp.zeros_like(kv, dtype=jnp.uint32))

    ltm = u < p
    c_lt = jnp.sum(ltm.astype(jnp.int32), axis=1, keepdims=True)
    masked = _vals_from_keys(u)
    sum_lt = jnp.sum(jnp.where(ltm, masked, 0.0), axis=1, keepdims=True)
    thr = _vals_from_keys(p)
    row_sel = sum_lt + (kv - c_lt).astype(jnp.float32) * thr
    row_sel = jnp.where(kv > 0, row_sel, 0.0)

    lane = lax.broadcasted_iota(jnp.int32, (1, 128), 1)
    out_ref[...] = jnp.where(lane == 0, jnp.sum(row_sel), 0.0)


@jax.jit
def kernel(bbox_input, label_input, bbox_target, label_target):
    B, A, _ = bbox_input.shape
    C = label_input.shape[1]
    lt = label_target.astype(jnp.int32)
    bb_in = jnp.transpose(bbox_input, (0, 2, 1))
    bb_tg = jnp.transpose(bbox_target, (0, 2, 1))

    keys, npr, acc = pl.pallas_call(
        _stage1_body,
        grid=(B // RB,),
        in_specs=[
            pl.BlockSpec((RB, 4, A), lambda b: (b, 0, 0)),
            pl.BlockSpec((RB, 4, A), lambda b: (b, 0, 0)),
            pl.BlockSpec((RB, C, A), lambda b: (b, 0, 0)),
            pl.BlockSpec((RB, A), lambda b: (b, 0)),
        ],
        out_specs=[
            pl.BlockSpec((RB, A), lambda b: (b, 0)),
            pl.BlockSpec((RB, 128), lambda b: (b, 0)),
            pl.BlockSpec((1, 128), lambda b: (0, 0)),
        ],
        out_shape=[
            jax.ShapeDtypeStruct((B, A), jnp.uint32),
            jax.ShapeDtypeStruct((B, 128), jnp.float32),
            jax.ShapeDtypeStruct((1, 128), jnp.float32),
        ],
        compiler_params=pltpu.CompilerParams(
            dimension_semantics=("arbitrary",),
        ),
    )(bb_in, bb_tg, label_input, lt)

    sel = pl.pallas_call(
        _stage2_body,
        in_specs=[
            pl.BlockSpec((B, A), lambda: (0, 0)),
            pl.BlockSpec((B, 128), lambda: (0, 0)),
        ],
        out_specs=pl.BlockSpec((1, 128), lambda: (0, 0)),
        out_shape=jax.ShapeDtypeStruct((1, 128), jnp.float32),
    )(keys, npr)

    sl1_pos, pos_ll, npos = acc[0, 0], acc[0, 1], acc[0, 2]
    return (sl1_pos + pos_ll - sel[0, 0]) / npos
